# baseline (speedup 1.0000x reference)
import jax
import jax.numpy as jnp
from jax import lax
from jax.experimental import pallas as pl
from jax.experimental.pallas import tpu as pltpu
from jax.experimental.pallas import tpu_sc as plsc

B, H, W = 4, 64, 64
R = 3.0
THRESH = 1.05
ROWS = 8
NCHUNK = 8
SLOTS = 4


def _sc_body(coords_hbm, noise_hbm, corr_hbm, out_hbm, halo_hbm,
             y_ext, x_ext, s_own, cy_b, cx_b, nz, idxb, rowbuf, sem):
    c = lax.axis_index("c")
    s = lax.axis_index("s")
    batch = 2 * c + s // 8
    blk = s % 8
    r0 = blk * ROWS
    wid = c * 16 + s

    lanes = lax.iota(jnp.int32, 16)

    pltpu.sync_copy(coords_hbm.at[batch, 0, pl.ds(r0, ROWS)],
                    y_ext.at[pl.ds(1, ROWS)])
    pltpu.sync_copy(coords_hbm.at[batch, 1, pl.ds(r0, ROWS)],
                    x_ext.at[pl.ds(1, ROWS)])
    for m in range(3):
        for pln in range(2):
            pltpu.sync_copy(noise_hbm.at[m, batch, pln, pl.ds(r0, ROWS)],
                            nz.at[m, pln])

    def exchange():
        pltpu.sync_copy(y_ext.at[1], halo_hbm.at[wid, 0])
        pltpu.sync_copy(x_ext.at[1], halo_hbm.at[wid, 1])
        pltpu.sync_copy(y_ext.at[ROWS], halo_hbm.at[wid, 2])
        pltpu.sync_copy(x_ext.at[ROWS], halo_hbm.at[wid, 3])
        plsc.subcore_barrier()
        sbase = c * 16 + (s // 8) * 8
        s_top = sbase + ((blk + 7) % 8)
        s_bot = sbase + ((blk + 1) % 8)
        pltpu.sync_copy(halo_hbm.at[s_top, 2], y_ext.at[0])
        pltpu.sync_copy(halo_hbm.at[s_top, 3], x_ext.at[0])
        pltpu.sync_copy(halo_hbm.at[s_bot, 0], y_ext.at[ROWS + 1])
        pltpu.sync_copy(halo_hbm.at[s_bot, 1], x_ext.at[ROWS + 1])
        plsc.subcore_barrier()

    def tail(r, b_, cy, cx):
        cb = b_ * 16
        cy_b[r, pl.ds(cb, 16)] = cy
        cx_b[r, pl.ds(cb, 16)] = cx
        col = cb + lanes
        n = batch * 4096 + (r0 + r) * 64 + col
        pairbase = n << 5
        y0 = cy.astype(jnp.int32)
        y1 = jnp.minimum(y0 + 1, H - 1)
        idxb[r, pl.ds(cb, 16)] = pairbase + (y0 >> 1)
        idxb[r, pl.ds(64 + cb, 16)] = pairbase + (y1 >> 1)

    def fire_dma(r):
        pltpu.async_copy(corr_hbm.at[idxb.at[r]],
                         rowbuf.at[pl.ds((r % SLOTS) * 128, 128)], sem)

    def fire4(h):
        def fb(r, _):
            fire_dma(r)
            return 0
        lax.fori_loop(h, h + SLOTS, fb, 0)

    def drain4(h):
        def dr(r, _):
            pltpu.make_async_copy(
                corr_hbm.at[idxb.at[r]],
                rowbuf.at[pl.ds((r % SLOTS) * 128, 128)], sem).wait()
            return 0
        lax.fori_loop(h, h + SLOTS, dr, 0)

    def drain_update(r, mode):
        for b_ in range(4):
            cb = b_ * 16
            cy = cy_b[r, pl.ds(cb, 16)]
            cx = cx_b[r, pl.ds(cb, 16)]
            y0 = cy.astype(jnp.int32)
            y1 = jnp.minimum(y0 + 1, H - 1)
            wy = cy - y0.astype(jnp.float32)
            x0 = cx.astype(jnp.int32)
            x1 = jnp.minimum(x0 + 1, W - 1)
            wx = cx - x0.astype(jnp.float32)
            base = (r % SLOTS) * 128 + cb + lanes
            c0 = (y0 & 1) * 64
            c1 = (y1 & 1) * 64
            v00 = plsc.load_gather(rowbuf, [base, c0 + x0])
            v01 = plsc.load_gather(rowbuf, [base, c0 + x1])
            v10 = plsc.load_gather(rowbuf, [base + 64, c1 + x0])
            v11 = plsc.load_gather(rowbuf, [base + 64, c1 + x1])
            sc = (v00 * (1.0 - wy) * (1.0 - wx) + v01 * (1.0 - wy) * wx
                  + v10 * wy * (1.0 - wx) + v11 * wy * wx)
            if mode == "init":
                s_own[r, pl.ds(cb, 16)] = sc
            else:
                sold = s_own[r, pl.ds(cb, 16)]
                if mode == "prop":
                    upd = sc > sold
                else:
                    upd = sc > jnp.float32(THRESH) * sold
                yold = y_ext[1 + r, pl.ds(cb, 16)]
                xold = x_ext[1 + r, pl.ds(cb, 16)]
                y_ext[1 + r, pl.ds(cb, 16)] = jnp.where(upd, cy, yold)
                x_ext[1 + r, pl.ds(cb, 16)] = jnp.where(upd, cx, xold)
                s_own[r, pl.ds(cb, 16)] = jnp.where(upd, sc, sold)

    def run_eval(candgen, mode):
        lax.fori_loop(0, NCHUNK, candgen, 0)
        fire4(0)
        drain4(0)

        def upd0(r, _):
            drain_update(r, mode)
            return 0
        lax.fori_loop(0, SLOTS, upd0, 0)
        fire4(SLOTS)
        drain4(SLOTS)
        lax.fori_loop(SLOTS, NCHUNK, upd0, 0)

    def eval_init():
        def candgen(r, _):
            for b_ in range(4):
                cb = b_ * 16
                cy = y_ext[1 + r, pl.ds(cb, 16)]
                cx = x_ext[1 + r, pl.ds(cb, 16)]
                tail(r, b_, cy, cx)
            return 0
        run_eval(candgen, "init")

    def eval_prop(dy, dx):
        exchange()

        def candgen(r, _):
            srow = jnp.broadcast_to(r + (1 - dy), (16,)).astype(jnp.int32)
            for b_ in range(4):
                cb = b_ * 16
                col = cb + lanes
                scol = (col - dx) & 63
                gy = plsc.load_gather(y_ext, [srow, scol])
                gx = plsc.load_gather(x_ext, [srow, scol])
                cy = jnp.minimum(jnp.maximum(gy + jnp.float32(dy), 0.0),
                                 jnp.float32(H - 1))
                cx = jnp.minimum(jnp.maximum(gx + jnp.float32(dx), 0.0),
                                 jnp.float32(W - 1))
                tail(r, b_, cy, cx)
            return 0
        run_eval(candgen, "prop")

    def eval_rand(m):
        def candgen(r, _):
            for b_ in range(4):
                cb = b_ * 16
                ny = y_ext[1 + r, pl.ds(cb, 16)] + nz[m, 0, r, pl.ds(cb, 16)]
                nx = x_ext[1 + r, pl.ds(cb, 16)] + nz[m, 1, r, pl.ds(cb, 16)]
                ny = jnp.where(ny < 0.0, 0.0, ny)
                nx = jnp.where(nx < 0.0, 0.0, nx)
                mh = ny >= jnp.float32(H)
                ny = jnp.where(mh, jnp.float32(H - 1), ny)
                nx = jnp.where(mh, jnp.float32(H - 1), nx)
                mw = nx >= jnp.float32(W)
                ny = jnp.where(mw, jnp.float32(W - 1), ny)
                nx = jnp.where(mw, jnp.float32(W - 1), nx)
                tail(r, b_, ny, nx)
            return 0
        run_eval(candgen, "rand")

    eval_init()
    eval_prop(1, 1)
    eval_rand(0)
    eval_prop(-1, -1)
    eval_rand(1)
    eval_prop(-1, 1)
    eval_rand(2)
    eval_prop(1, -1)

    pltpu.sync_copy(y_ext.at[pl.ds(1, ROWS)],
                    out_hbm.at[batch, 0, pl.ds(r0, ROWS)])
    pltpu.sync_copy(x_ext.at[pl.ds(1, ROWS)],
                    out_hbm.at[batch, 1, pl.ds(r0, ROWS)])


@jax.jit
def _run(raw_coords, noise_t, corr_pairs):
    mesh = plsc.VectorSubcoreMesh(core_axis_name="c", subcore_axis_name="s")
    f = pl.kernel(
        _sc_body,
        out_type=(jax.ShapeDtypeStruct((B, 2, H, W), jnp.float32),
                  jax.ShapeDtypeStruct((32, 4, W), jnp.float32)),
        mesh=mesh,
        compiler_params=pltpu.CompilerParams(needs_layout_passes=False),
        scratch_types=[
            pltpu.VMEM((ROWS + 2, W), jnp.float32),
            pltpu.VMEM((ROWS + 2, W), jnp.float32),
            pltpu.VMEM((ROWS, W), jnp.float32),
            pltpu.VMEM((ROWS, W), jnp.float32),
            pltpu.VMEM((ROWS, W), jnp.float32),
            pltpu.VMEM((3, 2, ROWS, W), jnp.float32),
            pltpu.VMEM((NCHUNK, 128), jnp.int32),
            pltpu.VMEM((SLOTS * 128, 128), jnp.float32),
            pltpu.SemaphoreType.DMA,
        ],
    )
    out, _halo = f(raw_coords, noise_t, corr_pairs)
    return out


def _detile_body(in_ref, out_ref):
    for q in range(32):
        out_ref[:, q, :] = jnp.concatenate(
            [in_ref[:, 2 * q, :], in_ref[:, 2 * q + 1, :]], axis=-1)


_DETILE_BM = 64


@jax.jit
def _detile(corr3):
    grid = (16384 // _DETILE_BM,)
    return pl.pallas_call(
        _detile_body,
        grid=grid,
        in_specs=[pl.BlockSpec((_DETILE_BM, 64, 64), lambda i: (i, 0, 0))],
        out_specs=pl.BlockSpec((_DETILE_BM, 32, 128), lambda i: (i, 0, 0)),
        out_shape=jax.ShapeDtypeStruct((16384, 32, 128), jnp.float32),
        compiler_params=pltpu.CompilerParams(
            dimension_semantics=("arbitrary",)),
    )(corr3)


def kernel(raw_coords, corr_map):
    key = jax.random.key(42)
    ks = jax.random.split(key, 3)
    noise = jnp.stack(
        [jax.random.normal(k, (B, H, W, 2), jnp.float32) * R for k in ks])
    noise_t = jnp.transpose(noise, (0, 1, 4, 2, 3))
    corr_pairs = _detile(corr_map.reshape(16384, 64, 64)).reshape(-1, 2 * W)
    return _run(raw_coords, noise_t, corr_pairs)


# speedup vs baseline: 1.3616x; 1.3616x over previous
import jax
import jax.numpy as jnp
from jax import lax
from jax.experimental import pallas as pl
from jax.experimental.pallas import tpu as pltpu
from jax.experimental.pallas import tpu_sc as plsc

B, H, W = 4, 64, 64
R = 3.0
THRESH = 1.05
ROWS = 8
NCHUNK = 8
SLOTS = 4


def _sc_body(coords_hbm, noise_hbm, corr_hbm, out_hbm, halo_hbm,
             y_ext, x_ext, s_own, cy_b, cx_b, nz, idxb, rowbuf, sem):
    c = lax.axis_index("c")
    s = lax.axis_index("s")
    batch = 2 * c + s // 8
    blk = s % 8
    r0 = blk * ROWS
    wid = c * 16 + s

    lanes = lax.iota(jnp.int32, 16)

    pltpu.sync_copy(coords_hbm.at[batch, 0, pl.ds(r0, ROWS)],
                    y_ext.at[pl.ds(1, ROWS)])
    pltpu.sync_copy(coords_hbm.at[batch, 1, pl.ds(r0, ROWS)],
                    x_ext.at[pl.ds(1, ROWS)])
    for m in range(3):
        for pln in range(2):
            pltpu.sync_copy(noise_hbm.at[m, batch, pln, pl.ds(r0, ROWS)],
                            nz.at[m, pln])

    def exchange():
        pltpu.sync_copy(y_ext.at[1], halo_hbm.at[wid, 0])
        pltpu.sync_copy(x_ext.at[1], halo_hbm.at[wid, 1])
        pltpu.sync_copy(y_ext.at[ROWS], halo_hbm.at[wid, 2])
        pltpu.sync_copy(x_ext.at[ROWS], halo_hbm.at[wid, 3])
        plsc.subcore_barrier()
        sbase = c * 16 + (s // 8) * 8
        s_top = sbase + ((blk + 7) % 8)
        s_bot = sbase + ((blk + 1) % 8)
        pltpu.sync_copy(halo_hbm.at[s_top, 2], y_ext.at[0])
        pltpu.sync_copy(halo_hbm.at[s_top, 3], x_ext.at[0])
        pltpu.sync_copy(halo_hbm.at[s_bot, 0], y_ext.at[ROWS + 1])
        pltpu.sync_copy(halo_hbm.at[s_bot, 1], x_ext.at[ROWS + 1])
        plsc.subcore_barrier()

    def tail(r, b_, cy, cx):
        cb = b_ * 16
        cy_b[r, pl.ds(cb, 16)] = cy
        cx_b[r, pl.ds(cb, 16)] = cx
        col = cb + lanes
        n = batch * 4096 + (r0 + r) * 64 + col
        rowbase = n << 6
        y0 = cy.astype(jnp.int32)
        y1 = jnp.minimum(y0 + 1, H - 1)
        idxb[r, pl.ds(cb, 16)] = rowbase + y0
        idxb[r, pl.ds(64 + cb, 16)] = rowbase + y1

    def fire_dma(r):
        pltpu.async_copy(corr_hbm.at[idxb.at[r]],
                         rowbuf.at[pl.ds((r % SLOTS) * 128, 128)], sem)

    def fire4(h):
        def fb(r, _):
            fire_dma(r)
            return 0
        lax.fori_loop(h, h + SLOTS, fb, 0)

    def drain4(h):
        def dr(r, _):
            pltpu.make_async_copy(
                corr_hbm.at[idxb.at[r]],
                rowbuf.at[pl.ds((r % SLOTS) * 128, 128)], sem).wait()
            return 0
        lax.fori_loop(h, h + SLOTS, dr, 0)

    def drain_update(r, mode):
        for b_ in range(4):
            cb = b_ * 16
            cy = cy_b[r, pl.ds(cb, 16)]
            cx = cx_b[r, pl.ds(cb, 16)]
            y0 = cy.astype(jnp.int32)
            y1 = jnp.minimum(y0 + 1, H - 1)
            wy = cy - y0.astype(jnp.float32)
            x0 = cx.astype(jnp.int32)
            x1 = jnp.minimum(x0 + 1, W - 1)
            wx = cx - x0.astype(jnp.float32)
            base = (r % SLOTS) * 128 + cb + lanes
            v00 = plsc.load_gather(rowbuf, [base, x0])
            v01 = plsc.load_gather(rowbuf, [base, x1])
            v10 = plsc.load_gather(rowbuf, [base + 64, x0])
            v11 = plsc.load_gather(rowbuf, [base + 64, x1])
            sc = (v00 * (1.0 - wy) * (1.0 - wx) + v01 * (1.0 - wy) * wx
                  + v10 * wy * (1.0 - wx) + v11 * wy * wx)
            if mode == "init":
                s_own[r, pl.ds(cb, 16)] = sc
            else:
                sold = s_own[r, pl.ds(cb, 16)]
                if mode == "prop":
                    upd = sc > sold
                else:
                    upd = sc > jnp.float32(THRESH) * sold
                yold = y_ext[1 + r, pl.ds(cb, 16)]
                xold = x_ext[1 + r, pl.ds(cb, 16)]
                y_ext[1 + r, pl.ds(cb, 16)] = jnp.where(upd, cy, yold)
                x_ext[1 + r, pl.ds(cb, 16)] = jnp.where(upd, cx, xold)
                s_own[r, pl.ds(cb, 16)] = jnp.where(upd, sc, sold)

    def run_eval(candgen, mode):
        lax.fori_loop(0, NCHUNK, candgen, 0)
        fire4(0)
        drain4(0)

        def upd0(r, _):
            drain_update(r, mode)
            return 0
        lax.fori_loop(0, SLOTS, upd0, 0)
        fire4(SLOTS)
        drain4(SLOTS)
        lax.fori_loop(SLOTS, NCHUNK, upd0, 0)

    def eval_init():
        def candgen(r, _):
            for b_ in range(4):
                cb = b_ * 16
                cy = y_ext[1 + r, pl.ds(cb, 16)]
                cx = x_ext[1 + r, pl.ds(cb, 16)]
                tail(r, b_, cy, cx)
            return 0
        run_eval(candgen, "init")

    def eval_prop(dy, dx):
        exchange()

        def candgen(r, _):
            srow = jnp.broadcast_to(r + (1 - dy), (16,)).astype(jnp.int32)
            for b_ in range(4):
                cb = b_ * 16
                col = cb + lanes
                scol = (col - dx) & 63
                gy = plsc.load_gather(y_ext, [srow, scol])
                gx = plsc.load_gather(x_ext, [srow, scol])
                cy = jnp.minimum(jnp.maximum(gy + jnp.float32(dy), 0.0),
                                 jnp.float32(H - 1))
                cx = jnp.minimum(jnp.maximum(gx + jnp.float32(dx), 0.0),
                                 jnp.float32(W - 1))
                tail(r, b_, cy, cx)
            return 0
        run_eval(candgen, "prop")

    def eval_rand(m):
        def candgen(r, _):
            for b_ in range(4):
                cb = b_ * 16
                ny = y_ext[1 + r, pl.ds(cb, 16)] + nz[m, 0, r, pl.ds(cb, 16)]
                nx = x_ext[1 + r, pl.ds(cb, 16)] + nz[m, 1, r, pl.ds(cb, 16)]
                ny = jnp.where(ny < 0.0, 0.0, ny)
                nx = jnp.where(nx < 0.0, 0.0, nx)
                mh = ny >= jnp.float32(H)
                ny = jnp.where(mh, jnp.float32(H - 1), ny)
                nx = jnp.where(mh, jnp.float32(H - 1), nx)
                mw = nx >= jnp.float32(W)
                ny = jnp.where(mw, jnp.float32(W - 1), ny)
                nx = jnp.where(mw, jnp.float32(W - 1), nx)
                tail(r, b_, ny, nx)
            return 0
        run_eval(candgen, "rand")

    eval_init()
    eval_prop(1, 1)
    eval_rand(0)
    eval_prop(-1, -1)
    eval_rand(1)
    eval_prop(-1, 1)
    eval_rand(2)
    eval_prop(1, -1)

    pltpu.sync_copy(y_ext.at[pl.ds(1, ROWS)],
                    out_hbm.at[batch, 0, pl.ds(r0, ROWS)])
    pltpu.sync_copy(x_ext.at[pl.ds(1, ROWS)],
                    out_hbm.at[batch, 1, pl.ds(r0, ROWS)])


@jax.jit
def _run(raw_coords, noise_t, corr_pairs):
    mesh = plsc.VectorSubcoreMesh(core_axis_name="c", subcore_axis_name="s")
    f = pl.kernel(
        _sc_body,
        out_type=(jax.ShapeDtypeStruct((B, 2, H, W), jnp.float32),
                  jax.ShapeDtypeStruct((32, 4, W), jnp.float32)),
        mesh=mesh,
        compiler_params=pltpu.CompilerParams(needs_layout_passes=False),
        scratch_types=[
            pltpu.VMEM((ROWS + 2, W), jnp.float32),
            pltpu.VMEM((ROWS + 2, W), jnp.float32),
            pltpu.VMEM((ROWS, W), jnp.float32),
            pltpu.VMEM((ROWS, W), jnp.float32),
            pltpu.VMEM((ROWS, W), jnp.float32),
            pltpu.VMEM((3, 2, ROWS, W), jnp.float32),
            pltpu.VMEM((NCHUNK, 128), jnp.int32),
            pltpu.VMEM((SLOTS * 128, 128), jnp.float32),
            pltpu.SemaphoreType.DMA,
        ],
    )
    out, _halo = f(raw_coords, noise_t, corr_pairs)
    return out


def _detile_body(in_ref, out_ref):
    x = in_ref[...]
    out_ref[...] = jnp.concatenate([x, x], axis=-1)


_DETILE_BM = 128


@jax.jit
def _detile(corr3):
    grid = (16384 // _DETILE_BM,)
    return pl.pallas_call(
        _detile_body,
        grid=grid,
        in_specs=[pl.BlockSpec((_DETILE_BM, 64, 64), lambda i: (i, 0, 0))],
        out_specs=pl.BlockSpec((_DETILE_BM, 64, 128), lambda i: (i, 0, 0)),
        out_shape=jax.ShapeDtypeStruct((16384, 64, 128), jnp.float32),
        compiler_params=pltpu.CompilerParams(
            dimension_semantics=("arbitrary",)),
    )(corr3)


def kernel(raw_coords, corr_map):
    key = jax.random.key(42)
    ks = jax.random.split(key, 3)
    noise = jnp.stack(
        [jax.random.normal(k, (B, H, W, 2), jnp.float32) * R for k in ks])
    noise_t = jnp.transpose(noise, (0, 1, 4, 2, 3))
    corr_pairs = _detile(corr_map.reshape(16384, 64, 64)).reshape(-1, 2 * W)
    return _run(raw_coords, noise_t, corr_pairs)


# speedup vs baseline: 1.3777x; 1.0118x over previous
import jax
import jax.numpy as jnp
from jax import lax
from jax.experimental import pallas as pl
from jax.experimental.pallas import tpu as pltpu
from jax.experimental.pallas import tpu_sc as plsc

B, H, W = 4, 64, 64
R = 3.0
THRESH = 1.05
ROWS = 8
NCHUNK = 8
SLOTS = 4


def _sc_body(coords_hbm, noise_hbm, corr_hbm, out_hbm, halo_hbm,
             y_ext, x_ext, s_own, cy_b, cx_b, nz, idxb, rowbuf, sem):
    c = lax.axis_index("c")
    s = lax.axis_index("s")
    batch = 2 * c + s // 8
    blk = s % 8
    r0 = blk * ROWS
    wid = c * 16 + s

    lanes = lax.iota(jnp.int32, 16)

    pltpu.sync_copy(coords_hbm.at[batch, 0, pl.ds(r0, ROWS)],
                    y_ext.at[pl.ds(1, ROWS)])
    pltpu.sync_copy(coords_hbm.at[batch, 1, pl.ds(r0, ROWS)],
                    x_ext.at[pl.ds(1, ROWS)])
    for m in range(3):
        for pln in range(2):
            pltpu.sync_copy(noise_hbm.at[m, batch, pln, pl.ds(r0, ROWS)],
                            nz.at[m, pln])

    def exchange():
        pltpu.sync_copy(y_ext.at[1], halo_hbm.at[wid, 0])
        pltpu.sync_copy(x_ext.at[1], halo_hbm.at[wid, 1])
        pltpu.sync_copy(y_ext.at[ROWS], halo_hbm.at[wid, 2])
        pltpu.sync_copy(x_ext.at[ROWS], halo_hbm.at[wid, 3])
        plsc.subcore_barrier()
        sbase = c * 16 + (s // 8) * 8
        s_top = sbase + ((blk + 7) % 8)
        s_bot = sbase + ((blk + 1) % 8)
        pltpu.sync_copy(halo_hbm.at[s_top, 2], y_ext.at[0])
        pltpu.sync_copy(halo_hbm.at[s_top, 3], x_ext.at[0])
        pltpu.sync_copy(halo_hbm.at[s_bot, 0], y_ext.at[ROWS + 1])
        pltpu.sync_copy(halo_hbm.at[s_bot, 1], x_ext.at[ROWS + 1])
        plsc.subcore_barrier()

    def tail(r, b_, cy, cx):
        cb = b_ * 16
        cy_b[r, pl.ds(cb, 16)] = cy
        cx_b[r, pl.ds(cb, 16)] = cx
        col = cb + lanes
        n = batch * 4096 + (r0 + r) * 64 + col
        rowbase = n << 6
        y0 = cy.astype(jnp.int32)
        y1 = jnp.minimum(y0 + 1, H - 1)
        idxb[r, pl.ds(cb, 16)] = rowbase + y0
        idxb[r, pl.ds(64 + cb, 16)] = rowbase + y1

    def fire_dma(r):
        pltpu.async_copy(corr_hbm.at[idxb.at[r]],
                         rowbuf.at[pl.ds((r % SLOTS) * 128, 128)], sem)

    def fire4(h):
        def fb(r, _):
            fire_dma(r)
            return 0
        lax.fori_loop(h, h + SLOTS, fb, 0)

    def drain4(h):
        def dr(r, _):
            pltpu.make_async_copy(
                corr_hbm.at[idxb.at[r]],
                rowbuf.at[pl.ds((r % SLOTS) * 128, 128)], sem).wait()
            return 0
        lax.fori_loop(h, h + SLOTS, dr, 0)

    def drain_update(r, mode):
        for b_ in range(4):
            cb = b_ * 16
            cy = cy_b[r, pl.ds(cb, 16)]
            cx = cx_b[r, pl.ds(cb, 16)]
            y0 = cy.astype(jnp.int32)
            y1 = jnp.minimum(y0 + 1, H - 1)
            wy = cy - y0.astype(jnp.float32)
            x0 = cx.astype(jnp.int32)
            x1 = jnp.minimum(x0 + 1, W - 1)
            wx = cx - x0.astype(jnp.float32)
            base = (r % SLOTS) * 128 + cb + lanes
            v00 = plsc.load_gather(rowbuf, [base, x0])
            v01 = plsc.load_gather(rowbuf, [base, x1])
            v10 = plsc.load_gather(rowbuf, [base + 64, x0])
            v11 = plsc.load_gather(rowbuf, [base + 64, x1])
            sc = (v00 * (1.0 - wy) * (1.0 - wx) + v01 * (1.0 - wy) * wx
                  + v10 * wy * (1.0 - wx) + v11 * wy * wx)
            if mode == "init":
                s_own[r, pl.ds(cb, 16)] = sc
            else:
                sold = s_own[r, pl.ds(cb, 16)]
                if mode == "prop":
                    upd = sc > sold
                else:
                    upd = sc > jnp.float32(THRESH) * sold
                yold = y_ext[1 + r, pl.ds(cb, 16)]
                xold = x_ext[1 + r, pl.ds(cb, 16)]
                y_ext[1 + r, pl.ds(cb, 16)] = jnp.where(upd, cy, yold)
                x_ext[1 + r, pl.ds(cb, 16)] = jnp.where(upd, cx, xold)
                s_own[r, pl.ds(cb, 16)] = jnp.where(upd, sc, sold)

    def run_eval(candgen, mode):
        lax.fori_loop(0, NCHUNK, candgen, 0)
        fire4(0)
        drain4(0)

        def upd0(r, _):
            drain_update(r, mode)
            return 0
        lax.fori_loop(0, SLOTS, upd0, 0)
        fire4(SLOTS)
        drain4(SLOTS)
        lax.fori_loop(SLOTS, NCHUNK, upd0, 0)

    def eval_init():
        def candgen(r, _):
            for b_ in range(4):
                cb = b_ * 16
                cy = y_ext[1 + r, pl.ds(cb, 16)]
                cx = x_ext[1 + r, pl.ds(cb, 16)]
                tail(r, b_, cy, cx)
            return 0
        run_eval(candgen, "init")

    def eval_prop(dy, dx):
        exchange()

        def candgen(r, _):
            srow = jnp.broadcast_to(r + (1 - dy), (16,)).astype(jnp.int32)
            for b_ in range(4):
                cb = b_ * 16
                col = cb + lanes
                scol = (col - dx) & 63
                gy = plsc.load_gather(y_ext, [srow, scol])
                gx = plsc.load_gather(x_ext, [srow, scol])
                cy = jnp.minimum(jnp.maximum(gy + jnp.float32(dy), 0.0),
                                 jnp.float32(H - 1))
                cx = jnp.minimum(jnp.maximum(gx + jnp.float32(dx), 0.0),
                                 jnp.float32(W - 1))
                tail(r, b_, cy, cx)
            return 0
        run_eval(candgen, "prop")

    def eval_rand(m):
        def candgen(r, _):
            for b_ in range(4):
                cb = b_ * 16
                ny = y_ext[1 + r, pl.ds(cb, 16)] + nz[m, 0, r, pl.ds(cb, 16)]
                nx = x_ext[1 + r, pl.ds(cb, 16)] + nz[m, 1, r, pl.ds(cb, 16)]
                ny = jnp.where(ny < 0.0, 0.0, ny)
                nx = jnp.where(nx < 0.0, 0.0, nx)
                mh = ny >= jnp.float32(H)
                ny = jnp.where(mh, jnp.float32(H - 1), ny)
                nx = jnp.where(mh, jnp.float32(H - 1), nx)
                mw = nx >= jnp.float32(W)
                ny = jnp.where(mw, jnp.float32(W - 1), ny)
                nx = jnp.where(mw, jnp.float32(W - 1), nx)
                tail(r, b_, ny, nx)
            return 0
        run_eval(candgen, "rand")

    eval_init()
    eval_prop(1, 1)
    eval_rand(0)
    eval_prop(-1, -1)
    eval_rand(1)
    eval_prop(-1, 1)
    eval_rand(2)
    eval_prop(1, -1)

    pltpu.sync_copy(y_ext.at[pl.ds(1, ROWS)],
                    out_hbm.at[batch, 0, pl.ds(r0, ROWS)])
    pltpu.sync_copy(x_ext.at[pl.ds(1, ROWS)],
                    out_hbm.at[batch, 1, pl.ds(r0, ROWS)])


@jax.jit
def _run(raw_coords, noise_t, corr_pairs):
    mesh = plsc.VectorSubcoreMesh(core_axis_name="c", subcore_axis_name="s")
    f = pl.kernel(
        _sc_body,
        out_type=(jax.ShapeDtypeStruct((B, 2, H, W), jnp.float32),
                  jax.ShapeDtypeStruct((32, 4, W), jnp.float32)),
        mesh=mesh,
        compiler_params=pltpu.CompilerParams(needs_layout_passes=False),
        scratch_types=[
            pltpu.VMEM((ROWS + 2, W), jnp.float32),
            pltpu.VMEM((ROWS + 2, W), jnp.float32),
            pltpu.VMEM((ROWS, W), jnp.float32),
            pltpu.VMEM((ROWS, W), jnp.float32),
            pltpu.VMEM((ROWS, W), jnp.float32),
            pltpu.VMEM((3, 2, ROWS, W), jnp.float32),
            pltpu.VMEM((NCHUNK, 128), jnp.int32),
            pltpu.VMEM((SLOTS * 128, 128), jnp.float32),
            pltpu.SemaphoreType.DMA,
        ],
    )
    out, _halo = f(raw_coords, noise_t, corr_pairs)
    return out


def _detile_body(in_ref, out_ref):
    x = in_ref[...]
    out_ref[...] = jnp.concatenate([x, x], axis=-1)


_DETILE_BM = 256


@jax.jit
def _detile(corr3):
    grid = (16384 // _DETILE_BM,)
    return pl.pallas_call(
        _detile_body,
        grid=grid,
        in_specs=[pl.BlockSpec((_DETILE_BM, 64, 64), lambda i: (i, 0, 0))],
        out_specs=pl.BlockSpec((_DETILE_BM, 64, 128), lambda i: (i, 0, 0)),
        out_shape=jax.ShapeDtypeStruct((16384, 64, 128), jnp.float32),
        compiler_params=pltpu.CompilerParams(
            dimension_semantics=("arbitrary",)),
    )(corr3)


def kernel(raw_coords, corr_map):
    key = jax.random.key(42)
    ks = jax.random.split(key, 3)
    noise = jnp.stack(
        [jax.random.normal(k, (B, H, W, 2), jnp.float32) * R for k in ks])
    noise_t = jnp.transpose(noise, (0, 1, 4, 2, 3))
    corr_pairs = _detile(corr_map.reshape(16384, 64, 64)).reshape(-1, 2 * W)
    return _run(raw_coords, noise_t, corr_pairs)
